# baseline (speedup 1.0000x reference)
import jax
import jax.numpy as jnp
from jax.experimental import pallas as pl


def _onehot_argmax_body(x_ref, o_ref):
    v = x_ref[...]
    p = v.shape[1]
    m = jnp.max(v, axis=1, keepdims=True)
    iota = jax.lax.broadcasted_iota(jnp.int32, v.shape, 1)
    masked = jnp.where(v == m, iota, p)
    mi = jnp.min(masked, axis=1, keepdims=True)
    o_ref[...] = (iota == mi).astype(v.dtype)


def kernel(x):
    b, t, p = x.shape
    rows = b * t
    xf = x.reshape(rows, p)
    blk = 2048
    grid = rows // blk
    y = pl.pallas_call(
        _onehot_argmax_body,
        grid=(grid,),
        in_specs=[pl.BlockSpec((blk, p), lambda i: (i, 0))],
        out_specs=pl.BlockSpec((blk, p), lambda i: (i, 0)),
        out_shape=jax.ShapeDtypeStruct((rows, p), x.dtype),
    )(xf)
    return y.reshape(b, t, p)
